# baseline (speedup 1.0000x reference)
import functools

import jax
import jax.numpy as jnp
from jax import lax
from jax.experimental import pallas as pl
from jax.experimental.pallas import tpu as pltpu
from jax.experimental.pallas import tpu_sc as plsc

_N = 10000
_E = 320000
_D = 128
_K = 128
_OH = 8


_TILES = 16
_SCORES = 2
_EPW = _E // (_SCORES * _TILES)
_C = 80
_NCH = _EPW // _C
_IB = 25
_NOUT = _NCH // _IB
_NBUF = 3
_NPAD = 10240
_RPT = _NPAD // _TILES
_ZR = 32


def _seg_sum_one(table, snd3, rcv3):
    mesh = plsc.VectorSubcoreMesh(core_axis_name="c", subcore_axis_name="s")

    @functools.partial(
        pl.kernel,
        mesh=mesh,
        out_type=jax.ShapeDtypeStruct((_SCORES, _NPAD, _D), jnp.float32),
        scratch_types=[
            pltpu.VMEM((_IB, _C), jnp.int32),
            pltpu.VMEM((_IB, _C), jnp.int32),
            pltpu.VMEM((_NBUF, _C, _D), jnp.float32),
            pltpu.VMEM((_ZR, _D), jnp.float32),
            pltpu.VMEM_SHARED((_NPAD, _D), jnp.float32),
            pltpu.SemaphoreType.DMA,
            pltpu.SemaphoreType.DMA,
            pltpu.SemaphoreType.DMA,
            pltpu.SemaphoreType.DMA,
            pltpu.SemaphoreType.DMA,
            pltpu.SemaphoreType.DMA,
        ],
    )
    def seg_kernel(tab_hbm, snd_hbm, rcv_hbm, parts_hbm,
                   snd_v, rcv_v, rows_v, zbuf, acc,
                   gsem0, gsem1, gsem2, ssem0, ssem1, ssem2):
        gsems = [gsem0, gsem1, gsem2]
        ssems = [ssem0, ssem1, ssem2]
        cid = lax.axis_index("c")
        sid = lax.axis_index("s")

        zv = jnp.zeros((16,), jnp.float32)
        for r in range(_ZR):
            for c8 in range(_D // 16):
                zbuf[r, pl.ds(c8 * 16, 16)] = zv

        base = sid * _RPT

        def zero_body(i, carry):
            pltpu.sync_copy(zbuf, acc.at[pl.ds(base + i * _ZR, _ZR)])
            return carry

        lax.fori_loop(0, _RPT // _ZR, zero_body, 0)

        plsc.subcore_barrier()

        def edge_phase(src_hbm):
            def plane_body(j, carry):
                pbase = (cid * _TILES + sid) * _NOUT + j
                pltpu.sync_copy(snd_hbm.at[pbase], snd_v)
                pltpu.sync_copy(rcv_hbm.at[pbase], rcv_v)
                gw = [None] * _NBUF
                sw = [None] * _NBUF
                for r in range(min(_NBUF - 1, _IB)):
                    gw[r] = pltpu.async_copy(
                        src_hbm.at[snd_v.at[r]], rows_v.at[r], gsems[r])
                for r in range(_IB):
                    b = r % _NBUF
                    gw[b].wait()
                    sw[b] = pltpu.async_copy(
                        rows_v.at[b], acc.at[rcv_v.at[r]], ssems[b],
                        add=True)
                    nxt = r + _NBUF - 1
                    if nxt < _IB:
                        nb = nxt % _NBUF
                        if sw[nb] is not None:
                            sw[nb].wait()
                        gw[nb] = pltpu.async_copy(
                            src_hbm.at[snd_v.at[nxt]], rows_v.at[nb],
                            gsems[nb])
                for b in range(_NBUF):
                    if sw[b] is not None:
                        sw[b].wait()
                return carry

            lax.fori_loop(0, _NOUT, plane_body, 0)

        edge_phase(tab_hbm)

        plsc.subcore_barrier()

        pltpu.sync_copy(acc.at[pl.ds(base, _RPT)],
                        parts_hbm.at[cid, pl.ds(base, _RPT)])

    return seg_kernel(table, snd3, rcv3)


_B = 512
_GRID1 = (_N + _B - 1) // _B
_CH = 16


def _bitonic_sort_rows(x):
    li = lax.broadcasted_iota(jnp.int32, (1, _K), 1)
    for k in (2, 4, 8, 16, 32, 64, 128):
        j = k // 2
        while j >= 1:
            lo = (li & j) == 0
            up = (li & k) == 0
            partner = jnp.where(lo, pltpu.roll(x, _K - j, 1), pltpu.roll(x, j, 1))
            mn = jnp.minimum(x, partner)
            mx = jnp.maximum(x, partner)
            x = jnp.where(lo == up, mn, mx)
            j //= 2
    return x


def _stage1a_body(ohp0_ref, ohp1_ref, oh_ref, wct_ref,
                  c1w_ref, c1b_ref, c2w_ref, c2b_ref,
                  zro_ref, newoh_ref, s_scr):
    new_oh = oh_ref[...] + (ohp0_ref[...] + ohp1_ref[...])
    newoh_ref[...] = new_oh

    s = _bitonic_sort_rows(new_oh)

    li = lax.broadcasted_iota(jnp.int32, (1, _K), 1)
    first = li == 0
    last = li == (_K - 1)

    def shift_pair(v):
        vm = jnp.where(first, 0.0, pltpu.roll(v, 1, 1))
        vp = jnp.where(last, 0.0, pltpu.roll(v, _K - 1, 1))
        return vm, vp

    s_scr[...] = s

    def chunk_body(i, carry):
        row0 = pl.multiple_of(i * _CH, _CH)
        sb = s_scr[pl.ds(row0, _CH), :]
        sbm, sbp = shift_pair(sb)
        h1 = []
        for c in range(8):
            hc = (c1w_ref[c, 0] * sbm + c1w_ref[c, 1] * sb
                  + c1w_ref[c, 2] * sbp + c1b_ref[c])
            h1.append(jnp.maximum(hc, 0.0))
        h1s = [shift_pair(hc) for hc in h1]
        zro = None
        for o in range(16):
            acc = jnp.full_like(sb, c2b_ref[o])
            for c in range(8):
                hm, hp = h1s[c]
                acc = acc + (c2w_ref[o, 3 * c] * hm
                             + c2w_ref[o, 3 * c + 1] * h1[c]
                             + c2w_ref[o, 3 * c + 2] * hp)
            acc = jnp.maximum(acc, 0.0)
            po = jnp.sum(acc, axis=1, keepdims=True)
            term = po * wct_ref[o:o + 1, :]
            zro = term if zro is None else zro + term
        zro_ref[pl.ds(row0, _CH), :] = zro
        return carry

    lax.fori_loop(0, _B // _CH, chunk_body, 0)


def _stage1a_call(ohp0, ohp1, oh, wct, c1w2, c1b, c2w2, c2b,
                  interpret=False):
    return pl.pallas_call(
        _stage1a_body,
        grid=(_GRID1,),
        in_specs=[
            pl.BlockSpec((_B, _K), lambda i: (i, 0)),
            pl.BlockSpec((_B, _K), lambda i: (i, 0)),
            pl.BlockSpec((_B, _K), lambda i: (i, 0)),
            pl.BlockSpec((16, _D), lambda i: (0, 0)),
            pl.BlockSpec(memory_space=pltpu.SMEM),
            pl.BlockSpec(memory_space=pltpu.SMEM),
            pl.BlockSpec(memory_space=pltpu.SMEM),
            pl.BlockSpec(memory_space=pltpu.SMEM),
        ],
        out_specs=[
            pl.BlockSpec((_B, _D), lambda i: (i, 0)),
            pl.BlockSpec((_B, _K), lambda i: (i, 0)),
        ],
        out_shape=[
            jax.ShapeDtypeStruct((_N, _D), jnp.float32),
            jax.ShapeDtypeStruct((_N, _K), jnp.float32),
        ],
        scratch_shapes=[pltpu.VMEM((_B, _K), jnp.float32)],
        interpret=interpret,
    )(ohp0, ohp1, oh, wct, c1w2, c1b, c2w2, c2b)


def _final_body(xp0_ref, xp1_ref, zro_ref, w1at_ref, beff_ref, w2t_ref,
                gamma_ref, beta_ref, b2_ref,
                out_ref, z_scr, sums_scr):
    p = pl.program_id(0)
    blk = pl.program_id(1)

    @pl.when(p == 0)
    def _():
        aggx = xp0_ref[...] + xp1_ref[...]
        z = jnp.dot(aggx, w1at_ref[...], preferred_element_type=jnp.float32)
        z = z + zro_ref[...] + beff_ref[...]
        z_scr[pl.ds(blk * _B, _B), :] = z

        rows = blk * _B + lax.broadcasted_iota(jnp.int32, (_B, 1), 0)
        zm = jnp.where(rows < _N, z, 0.0)

        @pl.when(blk == 0)
        def _():
            sums_scr[...] = jnp.zeros_like(sums_scr)

        sums_scr[0:1, :] += jnp.sum(zm, axis=0, keepdims=True)
        sums_scr[1:2, :] += jnp.sum(zm * zm, axis=0, keepdims=True)

    @pl.when(p == 1)
    def _():
        mu = sums_scr[0:1, :] * (1.0 / _N)
        ex2 = sums_scr[1:2, :] * (1.0 / _N)
        var = ex2 - mu * mu
        inv = lax.rsqrt(var + 1e-5)
        z = z_scr[pl.ds(blk * _B, _B), :]
        zn = (z - mu) * (inv * gamma_ref[...]) + beta_ref[...]
        zn = jnp.maximum(zn, 0.0)
        out_ref[...] = jnp.dot(zn, w2t_ref[...],
                               preferred_element_type=jnp.float32) + b2_ref[...]


def _final_call(xp0, xp1, zro, w1at, beff, w2t, gamma, beta, b2,
                interpret=False):
    last = _GRID1 - 1
    return pl.pallas_call(
        _final_body,
        grid=(2, _GRID1),
        in_specs=[
            pl.BlockSpec((_B, _D), lambda p, i: (jnp.where(p == 0, i, last), 0)),
            pl.BlockSpec((_B, _D), lambda p, i: (jnp.where(p == 0, i, last), 0)),
            pl.BlockSpec((_B, _D), lambda p, i: (jnp.where(p == 0, i, last), 0)),
            pl.BlockSpec((_D, _D), lambda p, i: (0, 0)),
            pl.BlockSpec((1, _D), lambda p, i: (0, 0)),
            pl.BlockSpec((_D, _D), lambda p, i: (0, 0)),
            pl.BlockSpec((1, _D), lambda p, i: (0, 0)),
            pl.BlockSpec((1, _D), lambda p, i: (0, 0)),
            pl.BlockSpec((1, _D), lambda p, i: (0, 0)),
        ],
        out_specs=pl.BlockSpec((_B, _D), lambda p, i: (i, 0)),
        out_shape=jax.ShapeDtypeStruct((_N, _D), jnp.float32),
        scratch_shapes=[
            pltpu.VMEM((_GRID1 * _B, _D), jnp.float32),
            pltpu.VMEM((8, _D), jnp.float32),
        ],
        interpret=interpret,
    )(xp0, xp1, zro, w1at, beff, w2t, gamma, beta, b2)


def kernel(xs, onehots, adjs, n_sample_nodes, W1, b1, gamma, beta, W2, b2,
           c1w, c1b, c2w, c2b, low, lob):
    x = xs[0]
    oh = onehots[0]
    adj = adjs[0].astype(jnp.int32)
    snd3 = adj[0].reshape(_SCORES * _TILES * _NOUT, _IB, _C)
    rcv3 = adj[1].reshape(_SCORES * _TILES * _NOUT, _IB, _C)

    ohparts = _seg_sum_one(oh, snd3, rcv3)
    xparts = _seg_sum_one(x, snd3, rcv3)

    W1a = W1[:, :_D]
    W1b = W1[:, _D:]
    wc = (W1b @ low) * (1.0 / _K)
    wct = wc.T
    beff = (b1 + W1b @ lob).reshape(1, _D)
    w1at = W1a.T
    c1w2 = c1w.reshape(8, 3)
    c2w2 = c2w.reshape(16, 24)

    zro, new_oh = _stage1a_call(
        ohparts[0], ohparts[1], oh, wct, c1w2, c1b, c2w2, c2b)

    z = _final_call(xparts[0], xparts[1], zro, w1at, beff, W2.T,
                    gamma.reshape(1, _D), beta.reshape(1, _D),
                    b2.reshape(1, _D))

    return (z[None], new_oh[None])


# speedup vs baseline: 1.3902x; 1.3902x over previous
import functools

import jax
import jax.numpy as jnp
from jax import lax
from jax.experimental import pallas as pl
from jax.experimental.pallas import tpu as pltpu
from jax.experimental.pallas import tpu_sc as plsc

_N = 10000
_E = 320000
_D = 128
_K = 128
_OH = 8


_TILES = 16
_SCORES = 2
_EPW = _E // (_SCORES * _TILES)
_C = 80
_NCH = _EPW // _C
_IB = 25
_NOUT = _NCH // _IB
_NBUF = 3
_NPAD = 10240
_RPT = _NPAD // _TILES
_ZR = 32


def _seg_sum_one(table, snd3, rcv3):
    mesh = plsc.VectorSubcoreMesh(core_axis_name="c", subcore_axis_name="s")

    @functools.partial(
        pl.kernel,
        mesh=mesh,
        out_type=jax.ShapeDtypeStruct((_SCORES, _NPAD, _D), jnp.float32),
        scratch_types=[
            pltpu.VMEM((_IB, _C), jnp.int32),
            pltpu.VMEM((_IB, _C), jnp.int32),
            pltpu.VMEM((_NBUF, _C, _D), jnp.float32),
            pltpu.VMEM((_ZR, _D), jnp.float32),
            pltpu.VMEM_SHARED((_NPAD, _D), jnp.float32),
            pltpu.SemaphoreType.DMA,
            pltpu.SemaphoreType.DMA,
            pltpu.SemaphoreType.DMA,
            pltpu.SemaphoreType.DMA,
            pltpu.SemaphoreType.DMA,
            pltpu.SemaphoreType.DMA,
        ],
    )
    def seg_kernel(tab_hbm, snd_hbm, rcv_hbm, parts_hbm,
                   snd_v, rcv_v, rows_v, zbuf, acc,
                   gsem0, gsem1, gsem2, ssem0, ssem1, ssem2):
        gsems = [gsem0, gsem1, gsem2]
        ssems = [ssem0, ssem1, ssem2]
        cid = lax.axis_index("c")
        sid = lax.axis_index("s")

        zv = jnp.zeros((16,), jnp.float32)
        for r in range(_ZR):
            for c8 in range(_D // 16):
                zbuf[r, pl.ds(c8 * 16, 16)] = zv

        base = sid * _RPT

        def zero_body(i, carry):
            pltpu.sync_copy(zbuf, acc.at[pl.ds(base + i * _ZR, _ZR)])
            return carry

        lax.fori_loop(0, _RPT // _ZR, zero_body, 0)

        plsc.subcore_barrier()

        def edge_phase(src_hbm):
            def plane_body(j, carry):
                pbase = (cid * _TILES + sid) * _NOUT + j
                pltpu.sync_copy(snd_hbm.at[pbase], snd_v)
                pltpu.sync_copy(rcv_hbm.at[pbase], rcv_v)
                gw = [None] * _NBUF
                sw = [None] * _NBUF
                for r in range(min(_NBUF - 1, _IB)):
                    gw[r] = pltpu.async_copy(
                        src_hbm.at[snd_v.at[r]], rows_v.at[r], gsems[r])
                for r in range(_IB):
                    b = r % _NBUF
                    gw[b].wait()
                    sw[b] = pltpu.async_copy(
                        rows_v.at[b], acc.at[rcv_v.at[r]], ssems[b],
                        add=True)
                    nxt = r + _NBUF - 1
                    if nxt < _IB:
                        nb = nxt % _NBUF
                        if sw[nb] is not None:
                            sw[nb].wait()
                        gw[nb] = pltpu.async_copy(
                            src_hbm.at[snd_v.at[nxt]], rows_v.at[nb],
                            gsems[nb])
                for b in range(_NBUF):
                    if sw[b] is not None:
                        sw[b].wait()
                return carry

            lax.fori_loop(0, _NOUT, plane_body, 0)

        edge_phase(tab_hbm)

        plsc.subcore_barrier()

        pltpu.sync_copy(acc.at[pl.ds(base, _RPT)],
                        parts_hbm.at[cid, pl.ds(base, _RPT)])

    return seg_kernel(table, snd3, rcv3)


_B = 512
_GRID1 = (_N + _B - 1) // _B
_CH = 16


def _bitonic_sort_rows(x):
    li = lax.broadcasted_iota(jnp.int32, (1, _K), 1)
    for k in (2, 4, 8, 16, 32, 64, 128):
        j = k // 2
        while j >= 1:
            lo = (li & j) == 0
            up = (li & k) == 0
            partner = jnp.where(lo, pltpu.roll(x, _K - j, 1), pltpu.roll(x, j, 1))
            mn = jnp.minimum(x, partner)
            mx = jnp.maximum(x, partner)
            x = jnp.where(lo == up, mn, mx)
            j //= 2
    return x


def _stage1a_body(ohp0_ref, ohp1_ref, oh_ref, wct_ref,
                  c1w_ref, c1b_ref, c2w_ref, c2b_ref,
                  zro_ref, newoh_ref):
    new_oh = oh_ref[...] + (ohp0_ref[...] + ohp1_ref[...])
    newoh_ref[...] = new_oh

    s = _bitonic_sort_rows(new_oh)

    li = lax.broadcasted_iota(jnp.int32, (1, _K), 1)
    first = li == 0
    last = li == (_K - 1)

    def shift_pair(v):
        vm = jnp.where(first, 0.0, pltpu.roll(v, 1, 1))
        vp = jnp.where(last, 0.0, pltpu.roll(v, _K - 1, 1))
        return vm, vp

    sm, sp = shift_pair(s)
    h1 = []
    for c in range(8):
        hc = (c1w_ref[c, 0] * sm + c1w_ref[c, 1] * s + c1w_ref[c, 2] * sp
              + c1b_ref[c])
        h1.append(jnp.maximum(hc, 0.0))
    h1s = [shift_pair(hc) for hc in h1]
    pooled = []
    for o in range(16):
        acc = jnp.full_like(s, c2b_ref[o])
        for c in range(8):
            hm, hp = h1s[c]
            acc = acc + (c2w_ref[o, 3 * c] * hm
                         + c2w_ref[o, 3 * c + 1] * h1[c]
                         + c2w_ref[o, 3 * c + 2] * hp)
        acc = jnp.maximum(acc, 0.0)
        pooled.append(jnp.sum(acc, axis=1, keepdims=True))

    zro = pooled[0] * wct_ref[0:1, :]
    for o in range(1, 16):
        zro = zro + pooled[o] * wct_ref[o:o + 1, :]
    zro_ref[...] = zro


def _stage1a_call(ohp0, ohp1, oh, wct, c1w2, c1b, c2w2, c2b,
                  interpret=False):
    return pl.pallas_call(
        _stage1a_body,
        grid=(_GRID1,),
        in_specs=[
            pl.BlockSpec((_B, _K), lambda i: (i, 0)),
            pl.BlockSpec((_B, _K), lambda i: (i, 0)),
            pl.BlockSpec((_B, _K), lambda i: (i, 0)),
            pl.BlockSpec((16, _D), lambda i: (0, 0)),
            pl.BlockSpec(memory_space=pltpu.SMEM),
            pl.BlockSpec(memory_space=pltpu.SMEM),
            pl.BlockSpec(memory_space=pltpu.SMEM),
            pl.BlockSpec(memory_space=pltpu.SMEM),
        ],
        out_specs=[
            pl.BlockSpec((_B, _D), lambda i: (i, 0)),
            pl.BlockSpec((_B, _K), lambda i: (i, 0)),
        ],
        out_shape=[
            jax.ShapeDtypeStruct((_N, _D), jnp.float32),
            jax.ShapeDtypeStruct((_N, _K), jnp.float32),
        ],
        interpret=interpret,
    )(ohp0, ohp1, oh, wct, c1w2, c1b, c2w2, c2b)


def _final_body(xp0_ref, xp1_ref, zro_ref, w1at_ref, beff_ref, w2t_ref,
                gamma_ref, beta_ref, b2_ref,
                out_ref, z_scr, sums_scr):
    p = pl.program_id(0)
    blk = pl.program_id(1)

    @pl.when(p == 0)
    def _():
        aggx = xp0_ref[...] + xp1_ref[...]
        z = jnp.dot(aggx, w1at_ref[...], preferred_element_type=jnp.float32)
        z = z + zro_ref[...] + beff_ref[...]
        z_scr[pl.ds(blk * _B, _B), :] = z

        rows = blk * _B + lax.broadcasted_iota(jnp.int32, (_B, 1), 0)
        zm = jnp.where(rows < _N, z, 0.0)

        @pl.when(blk == 0)
        def _():
            sums_scr[...] = jnp.zeros_like(sums_scr)

        sums_scr[0:1, :] += jnp.sum(zm, axis=0, keepdims=True)
        sums_scr[1:2, :] += jnp.sum(zm * zm, axis=0, keepdims=True)

    @pl.when(p == 1)
    def _():
        mu = sums_scr[0:1, :] * (1.0 / _N)
        ex2 = sums_scr[1:2, :] * (1.0 / _N)
        var = ex2 - mu * mu
        inv = lax.rsqrt(var + 1e-5)
        z = z_scr[pl.ds(blk * _B, _B), :]
        zn = (z - mu) * (inv * gamma_ref[...]) + beta_ref[...]
        zn = jnp.maximum(zn, 0.0)
        out_ref[...] = jnp.dot(zn, w2t_ref[...],
                               preferred_element_type=jnp.float32) + b2_ref[...]


def _final_call(xp0, xp1, zro, w1at, beff, w2t, gamma, beta, b2,
                interpret=False):
    last = _GRID1 - 1
    return pl.pallas_call(
        _final_body,
        grid=(2, _GRID1),
        in_specs=[
            pl.BlockSpec((_B, _D), lambda p, i: (jnp.where(p == 0, i, last), 0)),
            pl.BlockSpec((_B, _D), lambda p, i: (jnp.where(p == 0, i, last), 0)),
            pl.BlockSpec((_B, _D), lambda p, i: (jnp.where(p == 0, i, last), 0)),
            pl.BlockSpec((_D, _D), lambda p, i: (0, 0)),
            pl.BlockSpec((1, _D), lambda p, i: (0, 0)),
            pl.BlockSpec((_D, _D), lambda p, i: (0, 0)),
            pl.BlockSpec((1, _D), lambda p, i: (0, 0)),
            pl.BlockSpec((1, _D), lambda p, i: (0, 0)),
            pl.BlockSpec((1, _D), lambda p, i: (0, 0)),
        ],
        out_specs=pl.BlockSpec((_B, _D), lambda p, i: (i, 0)),
        out_shape=jax.ShapeDtypeStruct((_N, _D), jnp.float32),
        scratch_shapes=[
            pltpu.VMEM((_GRID1 * _B, _D), jnp.float32),
            pltpu.VMEM((8, _D), jnp.float32),
        ],
        interpret=interpret,
    )(xp0, xp1, zro, w1at, beff, w2t, gamma, beta, b2)


def kernel(xs, onehots, adjs, n_sample_nodes, W1, b1, gamma, beta, W2, b2,
           c1w, c1b, c2w, c2b, low, lob):
    x = xs[0]
    oh = onehots[0]
    adj = adjs[0].astype(jnp.int32)
    snd3 = adj[0].reshape(_SCORES * _TILES * _NOUT, _IB, _C)
    rcv3 = adj[1].reshape(_SCORES * _TILES * _NOUT, _IB, _C)

    ohparts = _seg_sum_one(oh, snd3, rcv3)
    xparts = _seg_sum_one(x, snd3, rcv3)

    W1a = W1[:, :_D]
    W1b = W1[:, _D:]
    wc = (W1b @ low) * (1.0 / _K)
    wct = wc.T
    beff = (b1 + W1b @ lob).reshape(1, _D)
    w1at = W1a.T
    c1w2 = c1w.reshape(8, 3)
    c2w2 = c2w.reshape(16, 24)

    zro, new_oh = _stage1a_call(
        ohparts[0], ohparts[1], oh, wct, c1w2, c1b, c2w2, c2b)

    z = _final_call(xparts[0], xparts[1], zro, w1at, beff, W2.T,
                    gamma.reshape(1, _D), beta.reshape(1, _D),
                    b2.reshape(1, _D))

    return (z[None], new_oh[None])


# speedup vs baseline: 1.4364x; 1.0332x over previous
import functools

import jax
import jax.numpy as jnp
from jax import lax
from jax.experimental import pallas as pl
from jax.experimental.pallas import tpu as pltpu
from jax.experimental.pallas import tpu_sc as plsc

_N = 10000
_E = 320000
_D = 128
_K = 128
_OH = 8


_TILES = 16
_SCORES = 2
_EPW = _E // (_SCORES * _TILES)
_C = 80
_NCH = _EPW // _C
_IB = 25
_NOUT = _NCH // _IB
_NBUF = 3
_NPAD = 10240
_RPT = _NPAD // _TILES
_ZR = 32


def _seg_sum_one(table, snd3, rcv3):
    mesh = plsc.VectorSubcoreMesh(core_axis_name="c", subcore_axis_name="s")

    @functools.partial(
        pl.kernel,
        mesh=mesh,
        out_type=jax.ShapeDtypeStruct((_SCORES, _NPAD, _D), jnp.float32),
        scratch_types=[
            pltpu.VMEM((_IB, _C), jnp.int32),
            pltpu.VMEM((_IB, _C), jnp.int32),
            pltpu.VMEM((_NBUF, _C, _D), jnp.float32),
            pltpu.VMEM((_ZR, _D), jnp.float32),
            pltpu.VMEM_SHARED((_NPAD, _D), jnp.float32),
            pltpu.SemaphoreType.DMA,
            pltpu.SemaphoreType.DMA,
            pltpu.SemaphoreType.DMA,
            pltpu.SemaphoreType.DMA,
            pltpu.SemaphoreType.DMA,
            pltpu.SemaphoreType.DMA,
        ],
    )
    def seg_kernel(tab_hbm, snd_hbm, rcv_hbm, parts_hbm,
                   snd_v, rcv_v, rows_v, zbuf, acc,
                   gsem0, gsem1, gsem2, ssem0, ssem1, ssem2):
        gsems = [gsem0, gsem1, gsem2]
        ssems = [ssem0, ssem1, ssem2]
        cid = lax.axis_index("c")
        sid = lax.axis_index("s")

        zv = jnp.zeros((16,), jnp.float32)
        for r in range(_ZR):
            for c8 in range(_D // 16):
                zbuf[r, pl.ds(c8 * 16, 16)] = zv

        base = sid * _RPT

        def zero_body(i, carry):
            pltpu.sync_copy(zbuf, acc.at[pl.ds(base + i * _ZR, _ZR)])
            return carry

        lax.fori_loop(0, _RPT // _ZR, zero_body, 0)

        plsc.subcore_barrier()

        def edge_phase(src_hbm):
            def plane_body(j, carry):
                pbase = (cid * _TILES + sid) * _NOUT + j
                pltpu.sync_copy(snd_hbm.at[pbase], snd_v)
                pltpu.sync_copy(rcv_hbm.at[pbase], rcv_v)
                gw = [None] * _NBUF
                sw = [None] * _NBUF
                for r in range(min(_NBUF - 1, _IB)):
                    gw[r] = pltpu.async_copy(
                        src_hbm.at[snd_v.at[r]], rows_v.at[r], gsems[r])
                for r in range(_IB):
                    b = r % _NBUF
                    gw[b].wait()
                    sw[b] = pltpu.async_copy(
                        rows_v.at[b], acc.at[rcv_v.at[r]], ssems[b],
                        add=True)
                    nxt = r + _NBUF - 1
                    if nxt < _IB:
                        nb = nxt % _NBUF
                        if sw[nb] is not None:
                            sw[nb].wait()
                        gw[nb] = pltpu.async_copy(
                            src_hbm.at[snd_v.at[nxt]], rows_v.at[nb],
                            gsems[nb])
                for b in range(_NBUF):
                    if sw[b] is not None:
                        sw[b].wait()
                return carry

            lax.fori_loop(0, _NOUT, plane_body, 0)

        edge_phase(tab_hbm)

        plsc.subcore_barrier()

        pltpu.sync_copy(acc.at[pl.ds(base, _RPT)],
                        parts_hbm.at[cid, pl.ds(base, _RPT)])

    return seg_kernel(table, snd3, rcv3)


_B = 512
_GRID1 = (_N + _B - 1) // _B
_CH = 16


def _bitonic_sort_rows(x):
    li = lax.broadcasted_iota(jnp.int32, (1, _K), 1)
    for k in (2, 4, 8, 16, 32, 64, 128):
        j = k // 2
        while j >= 1:
            lo = (li & j) == 0
            up = (li & k) == 0
            partner = jnp.where(lo, pltpu.roll(x, _K - j, 1), pltpu.roll(x, j, 1))
            mn = jnp.minimum(x, partner)
            mx = jnp.maximum(x, partner)
            x = jnp.where(lo == up, mn, mx)
            j //= 2
    return x


def _stage1a_body(ohp0_ref, ohp1_ref, oh_ref, wct_ref,
                  c1w_ref, c1b_ref, c2w_ref, c2b_ref,
                  zro_ref, newoh_ref):
    new_oh = oh_ref[...] + (ohp0_ref[0] + ohp1_ref[0])
    newoh_ref[...] = new_oh

    s = _bitonic_sort_rows(new_oh)

    li = lax.broadcasted_iota(jnp.int32, (1, _K), 1)
    first = li == 0
    last = li == (_K - 1)

    def shift_pair(v):
        vm = jnp.where(first, 0.0, pltpu.roll(v, 1, 1))
        vp = jnp.where(last, 0.0, pltpu.roll(v, _K - 1, 1))
        return vm, vp

    sm, sp = shift_pair(s)
    h1 = []
    for c in range(8):
        hc = (c1w_ref[c, 0] * sm + c1w_ref[c, 1] * s + c1w_ref[c, 2] * sp
              + c1b_ref[c])
        h1.append(jnp.maximum(hc, 0.0))
    h1s = [shift_pair(hc) for hc in h1]
    pooled = []
    for o in range(16):
        acc = jnp.full_like(s, c2b_ref[o])
        for c in range(8):
            hm, hp = h1s[c]
            acc = acc + (c2w_ref[o, 3 * c] * hm
                         + c2w_ref[o, 3 * c + 1] * h1[c]
                         + c2w_ref[o, 3 * c + 2] * hp)
        acc = jnp.maximum(acc, 0.0)
        pooled.append(jnp.sum(acc, axis=1, keepdims=True))

    zro = pooled[0] * wct_ref[0:1, :]
    for o in range(1, 16):
        zro = zro + pooled[o] * wct_ref[o:o + 1, :]
    zro_ref[...] = zro


def _stage1a_call(ohp0, ohp1, oh, wct, c1w2, c1b, c2w2, c2b,
                  interpret=False):
    return pl.pallas_call(
        _stage1a_body,
        grid=(_GRID1,),
        in_specs=[
            pl.BlockSpec((1, _B, _K), lambda i: (0, i, 0)),
            pl.BlockSpec((1, _B, _K), lambda i: (1, i, 0)),
            pl.BlockSpec((_B, _K), lambda i: (i, 0)),
            pl.BlockSpec((16, _D), lambda i: (0, 0)),
            pl.BlockSpec(memory_space=pltpu.SMEM),
            pl.BlockSpec(memory_space=pltpu.SMEM),
            pl.BlockSpec(memory_space=pltpu.SMEM),
            pl.BlockSpec(memory_space=pltpu.SMEM),
        ],
        out_specs=[
            pl.BlockSpec((_B, _D), lambda i: (i, 0)),
            pl.BlockSpec((_B, _K), lambda i: (i, 0)),
        ],
        out_shape=[
            jax.ShapeDtypeStruct((_N, _D), jnp.float32),
            jax.ShapeDtypeStruct((_N, _K), jnp.float32),
        ],
        interpret=interpret,
    )(ohp0, ohp1, oh, wct, c1w2, c1b, c2w2, c2b)


def _final_body(xp0_ref, xp1_ref, zro_ref, w1at_ref, beff_ref, w2t_ref,
                gamma_ref, beta_ref, b2_ref,
                out_ref, z_scr, sums_scr):
    p = pl.program_id(0)
    blk = pl.program_id(1)

    @pl.when(p == 0)
    def _():
        aggx = xp0_ref[0] + xp1_ref[0]
        z = jnp.dot(aggx, w1at_ref[...], preferred_element_type=jnp.float32)
        z = z + zro_ref[...] + beff_ref[...]
        z_scr[pl.ds(blk * _B, _B), :] = z

        rows = blk * _B + lax.broadcasted_iota(jnp.int32, (_B, 1), 0)
        zm = jnp.where(rows < _N, z, 0.0)

        @pl.when(blk == 0)
        def _():
            sums_scr[...] = jnp.zeros_like(sums_scr)

        sums_scr[0:1, :] += jnp.sum(zm, axis=0, keepdims=True)
        sums_scr[1:2, :] += jnp.sum(zm * zm, axis=0, keepdims=True)

    @pl.when(p == 1)
    def _():
        mu = sums_scr[0:1, :] * (1.0 / _N)
        ex2 = sums_scr[1:2, :] * (1.0 / _N)
        var = ex2 - mu * mu
        inv = lax.rsqrt(var + 1e-5)
        z = z_scr[pl.ds(blk * _B, _B), :]
        zn = (z - mu) * (inv * gamma_ref[...]) + beta_ref[...]
        zn = jnp.maximum(zn, 0.0)
        out_ref[...] = jnp.dot(zn, w2t_ref[...],
                               preferred_element_type=jnp.float32) + b2_ref[...]


def _final_call(xp0, xp1, zro, w1at, beff, w2t, gamma, beta, b2,
                interpret=False):
    last = _GRID1 - 1
    return pl.pallas_call(
        _final_body,
        grid=(2, _GRID1),
        in_specs=[
            pl.BlockSpec((1, _B, _D),
                         lambda p, i: (0, jnp.where(p == 0, i, last), 0)),
            pl.BlockSpec((1, _B, _D),
                         lambda p, i: (1, jnp.where(p == 0, i, last), 0)),
            pl.BlockSpec((_B, _D), lambda p, i: (jnp.where(p == 0, i, last), 0)),
            pl.BlockSpec((_D, _D), lambda p, i: (0, 0)),
            pl.BlockSpec((1, _D), lambda p, i: (0, 0)),
            pl.BlockSpec((_D, _D), lambda p, i: (0, 0)),
            pl.BlockSpec((1, _D), lambda p, i: (0, 0)),
            pl.BlockSpec((1, _D), lambda p, i: (0, 0)),
            pl.BlockSpec((1, _D), lambda p, i: (0, 0)),
        ],
        out_specs=pl.BlockSpec((_B, _D), lambda p, i: (i, 0)),
        out_shape=jax.ShapeDtypeStruct((_N, _D), jnp.float32),
        scratch_shapes=[
            pltpu.VMEM((_GRID1 * _B, _D), jnp.float32),
            pltpu.VMEM((8, _D), jnp.float32),
        ],
        interpret=interpret,
    )(xp0, xp1, zro, w1at, beff, w2t, gamma, beta, b2)


def kernel(xs, onehots, adjs, n_sample_nodes, W1, b1, gamma, beta, W2, b2,
           c1w, c1b, c2w, c2b, low, lob):
    x = xs[0]
    oh = onehots[0]
    adj = adjs[0].astype(jnp.int32)
    snd3 = adj[0].reshape(_SCORES * _TILES * _NOUT, _IB, _C)
    rcv3 = adj[1].reshape(_SCORES * _TILES * _NOUT, _IB, _C)

    ohparts = _seg_sum_one(oh, snd3, rcv3)
    xparts = _seg_sum_one(x, snd3, rcv3)

    W1a = W1[:, :_D]
    W1b = W1[:, _D:]
    wc = (W1b @ low) * (1.0 / _K)
    wct = wc.T
    beff = (b1 + W1b @ lob).reshape(1, _D)
    w1at = W1a.T
    c1w2 = c1w.reshape(8, 3)
    c2w2 = c2w.reshape(16, 24)

    zro, new_oh = _stage1a_call(
        ohparts, ohparts, oh, wct, c1w2, c1b, c2w2, c2b)

    z = _final_call(xparts, xparts, zro, w1at, beff, W2.T,
                    gamma.reshape(1, _D), beta.reshape(1, _D),
                    b2.reshape(1, _D))

    return (z[None], new_oh[None])
